# baseline (speedup 1.0000x reference)
import functools

import jax
import jax.numpy as jnp
from jax import lax
from jax.experimental import pallas as pl
from jax.experimental.pallas import tpu as pltpu
from jax.experimental.pallas import tpu_sc as plsc

N = 4096
K = 16
RB = 256
GRID = N // RB
NEG_CAP = -3.4028234663852886e38


CB = 256
NCB = N // CB


def _knn_body(seg_ref, x_ref, xt_ref, brow_ref, bcol_ref, idx_ref,
              runv_ref, runi_ref):
    i = pl.program_id(0)
    j = pl.program_id(1)

    @pl.when(j == 0)
    def _init():
        runv_ref[...] = jnp.full((RB, K), -jnp.inf, jnp.float32)
        runi_ref[...] = N + lax.broadcasted_iota(jnp.int32, (RB, K), 1)

    c0 = seg_ref[i]
    c1 = seg_ref[GRID + i]
    active = (j * CB < c1) & (j * CB + CB > c0)

    @pl.when(active)
    def _tile():
        xb = x_ref[...]
        xt = xt_ref[...]
        sq_r = jnp.sum(xb * xb, axis=1, keepdims=True)
        sq_c = jnp.sum(xt * xt, axis=0, keepdims=True)
        prod = jnp.dot(xb, xt, preferred_element_type=jnp.float32)
        dist = (sq_r + sq_c) - 2.0 * prod
        col = j * CB + lax.broadcasted_iota(jnp.int32, (RB, CB), 1)
        row = i * RB + lax.broadcasted_iota(jnp.int32, (RB, CB), 0)
        valid = (brow_ref[...] == bcol_ref[...]) & (col != row)
        nd = jnp.where(valid, -dist, NEG_CAP)
        rv = runv_ref[...]
        ri = runi_ref[...]
        for t in range(K):
            m = jnp.maximum(jnp.max(nd, axis=1, keepdims=True),
                            jnp.max(rv, axis=1, keepdims=True))
            cn = jnp.min(jnp.where(nd == m, col, 2 * N), axis=1, keepdims=True)
            cr = jnp.min(jnp.where(rv == m, ri, 2 * N), axis=1, keepdims=True)
            am = jnp.minimum(cn, cr)
            runv_ref[:, t:t + 1] = m
            runi_ref[:, t:t + 1] = am
            nd = jnp.where(col == am, -jnp.inf, nd)
            rv = jnp.where(ri == am, -jnp.inf, rv)

    @pl.when(j == NCB - 1)
    def _emit():
        idx_ref[...] = runi_ref[...]


def _knn(xf, xt, segs, brow, bcol, d):
    grid_spec = pltpu.PrefetchScalarGridSpec(
        num_scalar_prefetch=1,
        grid=(GRID, NCB),
        in_specs=[
            pl.BlockSpec((RB, d), lambda i, j, s: (i, 0)),
            pl.BlockSpec((d, CB), lambda i, j, s: (0, j)),
            pl.BlockSpec((RB, 1), lambda i, j, s: (i, 0)),
            pl.BlockSpec((1, CB), lambda i, j, s: (0, j)),
        ],
        out_specs=pl.BlockSpec((RB, K), lambda i, j, s: (i, 0)),
        scratch_shapes=[pltpu.VMEM((RB, K), jnp.float32),
                        pltpu.VMEM((RB, K), jnp.int32)],
    )
    return pl.pallas_call(
        _knn_body,
        grid_spec=grid_spec,
        out_shape=jax.ShapeDtypeStruct((N, K), jnp.int32),
    )(segs, xf, xt, brow, bcol)


_SC_CH = 128


@functools.cache
def _make_gather(d):
    info = plsc.get_sparse_core_info()
    nw = info.num_cores * info.num_subcores
    b_per_w = (N * K) // nw
    n_chunks = b_per_w // _SC_CH
    mesh = plsc.VectorSubcoreMesh(core_axis_name="c", subcore_axis_name="s")

    n_pairs = n_chunks // 2

    @functools.partial(
        pl.kernel,
        mesh=mesh,
        out_type=jax.ShapeDtypeStruct((N * K, d), jnp.float32),
        scratch_types=[
            pltpu.VMEM((b_per_w,), jnp.int32),
            pltpu.VMEM((_SC_CH, d), jnp.float32),
            pltpu.VMEM((_SC_CH, d), jnp.float32),
            pltpu.SemaphoreType.DMA,
            pltpu.SemaphoreType.DMA,
            pltpu.SemaphoreType.DMA,
            pltpu.SemaphoreType.DMA,
        ],
    )
    def gk(table_hbm, idx_hbm, out_hbm, idx_v, rows0, rows1,
           sem_g0, sem_g1, sem_w0, sem_w1):
        wid = lax.axis_index("s") * info.num_cores + lax.axis_index("c")
        base = wid * b_per_w
        pltpu.sync_copy(idx_hbm.at[pl.ds(base, b_per_w)], idx_v)

        def gsl(c):
            return table_hbm.at[idx_v.at[pl.ds(c * _SC_CH, _SC_CH)]]

        def osl(c):
            return out_hbm.at[pl.ds(base + c * _SC_CH, _SC_CH)]

        pltpu.async_copy(gsl(0), rows0, sem_g0)
        pltpu.async_copy(gsl(1), rows1, sem_g1)

        def body(p, carry):
            c = 2 * p
            pltpu.make_async_copy(gsl(c), rows0, sem_g0).wait()
            pltpu.async_copy(rows0, osl(c), sem_w0)
            pltpu.make_async_copy(gsl(c + 1), rows1, sem_g1).wait()
            pltpu.async_copy(rows1, osl(c + 1), sem_w1)

            @pl.when(p < n_pairs - 1)
            def _next():
                pltpu.make_async_copy(rows0, osl(c), sem_w0).wait()
                pltpu.async_copy(gsl(c + 2), rows0, sem_g0)
                pltpu.make_async_copy(rows1, osl(c + 1), sem_w1).wait()
                pltpu.async_copy(gsl(c + 3), rows1, sem_g1)

            return carry

        lax.fori_loop(0, n_pairs, body, 0)
        pltpu.make_async_copy(rows0, osl(2 * n_pairs - 2), sem_w0).wait()
        pltpu.make_async_copy(rows1, osl(2 * n_pairs - 1), sem_w1).wait()

    return gk


def _gather_rows(table, flat_idx):
    return _make_gather(table.shape[1])(table, flat_idx)


def _conv_body(x_ref, xj_ref, wa_ref, ba_ref, wb_ref, bb_ref, h_ref):
    xb = x_ref[...]
    d = xb.shape[1]
    wa = wa_ref[...]
    wb = wb_ref[...]
    ba = ba_ref[...]
    acc = jnp.full((RB, wb.shape[1]), -jnp.inf, jnp.float32)
    for t in range(K):
        xj = xj_ref[t][:, :d]
        e = jnp.concatenate([xb, xj - xb], axis=1)
        pre = jnp.maximum(jnp.dot(e, wa, preferred_element_type=jnp.float32) + ba, 0.0)
        acc = jnp.maximum(acc, jnp.dot(pre, wb, preferred_element_type=jnp.float32))
    h_ref[...] = acc + bb_ref[...]


def _conv(xf, xj, wa, ba, wb, bb, d, dg, emb):
    return pl.pallas_call(
        _conv_body,
        grid=(GRID,),
        in_specs=[
            pl.BlockSpec((RB, d), lambda i: (i, 0)),
            pl.BlockSpec((K, RB, dg), lambda i: (0, i, 0)),
            pl.BlockSpec((2 * d, emb), lambda i: (0, 0)),
            pl.BlockSpec((1, emb), lambda i: (0, 0)),
            pl.BlockSpec((emb, emb), lambda i: (0, 0)),
            pl.BlockSpec((1, emb), lambda i: (0, 0)),
        ],
        out_specs=pl.BlockSpec((RB, emb), lambda i: (i, 0)),
        out_shape=jax.ShapeDtypeStruct((N, emb), jnp.float32),
    )(xf, xj, wa, ba, wb, bb)


def _final_body(h_ref, hj_ref, w1_ref, b1_ref, w2_ref, b2_ref, out_ref):
    hd = h_ref[...]
    w1 = w1_ref[...]
    b1 = b1_ref[...]
    w2 = w2_ref[...]
    b2 = b2_ref[...]
    for t in range(K):
        hs = hj_ref[t]
        e = jnp.concatenate([hs, hd], axis=1)
        pre = jnp.maximum(jnp.dot(e, w1, preferred_element_type=jnp.float32) + b1, 0.0)
        lg = jnp.dot(pre, w2, preferred_element_type=jnp.float32) + b2
        out_ref[:, t:t + 1] = jax.nn.sigmoid(lg)


def _final(h, hj, w1, b1, w2, b2, emb, hid):
    return pl.pallas_call(
        _final_body,
        grid=(GRID,),
        in_specs=[
            pl.BlockSpec((RB, emb), lambda i: (i, 0)),
            pl.BlockSpec((K, RB, emb), lambda i: (0, i, 0)),
            pl.BlockSpec((2 * emb, hid), lambda i: (0, 0)),
            pl.BlockSpec((1, hid), lambda i: (0, 0)),
            pl.BlockSpec((hid, 1), lambda i: (0, 0)),
            pl.BlockSpec((1, 1), lambda i: (0, 0)),
        ],
        out_specs=pl.BlockSpec((RB, K), lambda i: (i, 0)),
        out_shape=jax.ShapeDtypeStruct((N, K), jnp.float32),
    )(h, hj, w1, b1, w2, b2)


def _edge_conv_layer(xf, segs, brow, bcol, wa, ba, wb, bb):
    d = xf.shape[1]
    emb = wb.shape[1]
    idx = _knn(xf, xf.T, segs, brow, bcol, d)
    table = xf if d % 128 == 0 else jnp.concatenate(
        [xf, jnp.zeros((N, 128 - d % 128), jnp.float32)], axis=1)
    dg = table.shape[1]
    xj = _gather_rows(table, idx.T.reshape(-1))
    return _conv(xf, xj.reshape(K, N, dg), wa, ba.reshape(1, emb),
                 wb, bb.reshape(1, emb), d, dg, emb)


def kernel(x, batch, W1a, b1a, W1b, b1b, W2a, b2a, W2b, b2b, Wm1, bm1, Wm2, bm2):
    brow = batch.reshape(N, 1)
    bcol = batch.reshape(1, N)
    starts = jnp.searchsorted(batch, batch[::RB]).astype(jnp.int32)
    ends = jnp.searchsorted(batch, batch[RB - 1::RB], side='right').astype(jnp.int32)
    segs = jnp.concatenate([starts, ends])
    h = _edge_conv_layer(x, segs, brow, bcol, W1a, b1a, W1b, b1b)
    h = _edge_conv_layer(h, segs, brow, bcol, W2a, b2a, W2b, b2b)
    emb = h.shape[1]
    hid = Wm1.shape[1]
    idx = _knn(h, h.T, segs, brow, bcol, emb)
    hj = _gather_rows(h, idx.T.reshape(-1))
    probs = _final(h, hj.reshape(K, N, emb), Wm1, bm1.reshape(1, hid),
                   Wm2, bm2.reshape(1, 1), emb, hid)
    link_probs = probs.reshape(N * K, 1)
    src = idx.reshape(-1)
    dst = jnp.repeat(jnp.arange(N, dtype=jnp.int32), K)
    edge_index = jnp.stack([src, dst], axis=0)
    return link_probs, edge_index


# speedup vs baseline: 1.6499x; 1.6499x over previous
import functools

import jax
import jax.numpy as jnp
from jax import lax
from jax.experimental import pallas as pl
from jax.experimental.pallas import tpu as pltpu
from jax.experimental.pallas import tpu_sc as plsc

N = 4096
K = 16
RB = 256
GRID = N // RB
NEG_CAP = -3.4028234663852886e38


CB = 1024
NCB = N // CB


def _knn_body(seg_ref, x_ref, xt_ref, brow_ref, bcol_ref, idx_ref,
              runv_ref, runi_ref):
    i = pl.program_id(0)
    j = pl.program_id(1)

    @pl.when(j == 0)
    def _init():
        runv_ref[...] = jnp.full((RB, K), -jnp.inf, jnp.float32)
        runi_ref[...] = N + lax.broadcasted_iota(jnp.int32, (RB, K), 1)

    c0 = seg_ref[i]
    c1 = seg_ref[GRID + i]
    active = (j * CB < c1) & (j * CB + CB > c0)

    @pl.when(active)
    def _tile():
        xb = x_ref[...]
        xt = xt_ref[...]
        sq_r = jnp.sum(xb * xb, axis=1, keepdims=True)
        sq_c = jnp.sum(xt * xt, axis=0, keepdims=True)
        prod = jnp.dot(xb, xt, preferred_element_type=jnp.float32)
        dist = (sq_r + sq_c) - 2.0 * prod
        col = j * CB + lax.broadcasted_iota(jnp.int32, (RB, CB), 1)
        row = i * RB + lax.broadcasted_iota(jnp.int32, (RB, CB), 0)
        valid = (brow_ref[...] == bcol_ref[...]) & (col != row)
        nd = jnp.where(valid, -dist, NEG_CAP)
        rv = runv_ref[...]
        ri = runi_ref[...]
        for t in range(K):
            m = jnp.maximum(jnp.max(nd, axis=1, keepdims=True),
                            jnp.max(rv, axis=1, keepdims=True))
            cn = jnp.min(jnp.where(nd == m, col, 2 * N), axis=1, keepdims=True)
            cr = jnp.min(jnp.where(rv == m, ri, 2 * N), axis=1, keepdims=True)
            am = jnp.minimum(cn, cr)
            runv_ref[:, t:t + 1] = m
            runi_ref[:, t:t + 1] = am
            nd = jnp.where(col == am, -jnp.inf, nd)
            rv = jnp.where(ri == am, -jnp.inf, rv)

    @pl.when(j == NCB - 1)
    def _emit():
        idx_ref[...] = runi_ref[...]


def _knn(xf, xt, segs, brow, bcol, d):
    grid_spec = pltpu.PrefetchScalarGridSpec(
        num_scalar_prefetch=1,
        grid=(GRID, NCB),
        in_specs=[
            pl.BlockSpec((RB, d), lambda i, j, s: (i, 0)),
            pl.BlockSpec((d, CB), lambda i, j, s: (0, j)),
            pl.BlockSpec((RB, 1), lambda i, j, s: (i, 0)),
            pl.BlockSpec((1, CB), lambda i, j, s: (0, j)),
        ],
        out_specs=pl.BlockSpec((RB, K), lambda i, j, s: (i, 0)),
        scratch_shapes=[pltpu.VMEM((RB, K), jnp.float32),
                        pltpu.VMEM((RB, K), jnp.int32)],
    )
    return pl.pallas_call(
        _knn_body,
        grid_spec=grid_spec,
        out_shape=jax.ShapeDtypeStruct((N, K), jnp.int32),
    )(segs, xf, xt, brow, bcol)


_SC_CH = 128


@functools.cache
def _make_gather(d):
    info = plsc.get_sparse_core_info()
    nw = info.num_cores * info.num_subcores
    b_per_w = (N * K) // nw
    n_chunks = b_per_w // _SC_CH
    mesh = plsc.VectorSubcoreMesh(core_axis_name="c", subcore_axis_name="s")

    n_pairs = n_chunks // 2

    @functools.partial(
        pl.kernel,
        mesh=mesh,
        out_type=jax.ShapeDtypeStruct((N * K, d), jnp.float32),
        scratch_types=[
            pltpu.VMEM((b_per_w,), jnp.int32),
            pltpu.VMEM((_SC_CH, d), jnp.float32),
            pltpu.VMEM((_SC_CH, d), jnp.float32),
            pltpu.SemaphoreType.DMA,
            pltpu.SemaphoreType.DMA,
            pltpu.SemaphoreType.DMA,
            pltpu.SemaphoreType.DMA,
        ],
    )
    def gk(table_hbm, idx_hbm, out_hbm, idx_v, rows0, rows1,
           sem_g0, sem_g1, sem_w0, sem_w1):
        wid = lax.axis_index("s") * info.num_cores + lax.axis_index("c")
        base = wid * b_per_w
        pltpu.sync_copy(idx_hbm.at[pl.ds(base, b_per_w)], idx_v)

        def gsl(c):
            return table_hbm.at[idx_v.at[pl.ds(c * _SC_CH, _SC_CH)]]

        def osl(c):
            return out_hbm.at[pl.ds(base + c * _SC_CH, _SC_CH)]

        pltpu.async_copy(gsl(0), rows0, sem_g0)
        pltpu.async_copy(gsl(1), rows1, sem_g1)

        def body(p, carry):
            c = 2 * p
            pltpu.make_async_copy(gsl(c), rows0, sem_g0).wait()
            pltpu.async_copy(rows0, osl(c), sem_w0)
            pltpu.make_async_copy(gsl(c + 1), rows1, sem_g1).wait()
            pltpu.async_copy(rows1, osl(c + 1), sem_w1)

            @pl.when(p < n_pairs - 1)
            def _next():
                pltpu.make_async_copy(rows0, osl(c), sem_w0).wait()
                pltpu.async_copy(gsl(c + 2), rows0, sem_g0)
                pltpu.make_async_copy(rows1, osl(c + 1), sem_w1).wait()
                pltpu.async_copy(gsl(c + 3), rows1, sem_g1)

            return carry

        lax.fori_loop(0, n_pairs, body, 0)
        pltpu.make_async_copy(rows0, osl(2 * n_pairs - 2), sem_w0).wait()
        pltpu.make_async_copy(rows1, osl(2 * n_pairs - 1), sem_w1).wait()

    return gk


def _gather_rows(table, flat_idx):
    return _make_gather(table.shape[1])(table, flat_idx)


def _conv_body(x_ref, xj_ref, wa_ref, ba_ref, wb_ref, bb_ref, h_ref):
    xb = x_ref[...]
    d = xb.shape[1]
    wa = wa_ref[...]
    wb = wb_ref[...]
    ba = ba_ref[...]
    acc = jnp.full((RB, wb.shape[1]), -jnp.inf, jnp.float32)
    for t in range(K):
        xj = xj_ref[t][:, :d]
        e = jnp.concatenate([xb, xj - xb], axis=1)
        pre = jnp.maximum(jnp.dot(e, wa, preferred_element_type=jnp.float32) + ba, 0.0)
        acc = jnp.maximum(acc, jnp.dot(pre, wb, preferred_element_type=jnp.float32))
    h_ref[...] = acc + bb_ref[...]


def _conv(xf, xj, wa, ba, wb, bb, d, dg, emb):
    return pl.pallas_call(
        _conv_body,
        grid=(GRID,),
        in_specs=[
            pl.BlockSpec((RB, d), lambda i: (i, 0)),
            pl.BlockSpec((K, RB, dg), lambda i: (0, i, 0)),
            pl.BlockSpec((2 * d, emb), lambda i: (0, 0)),
            pl.BlockSpec((1, emb), lambda i: (0, 0)),
            pl.BlockSpec((emb, emb), lambda i: (0, 0)),
            pl.BlockSpec((1, emb), lambda i: (0, 0)),
        ],
        out_specs=pl.BlockSpec((RB, emb), lambda i: (i, 0)),
        out_shape=jax.ShapeDtypeStruct((N, emb), jnp.float32),
    )(xf, xj, wa, ba, wb, bb)


def _final_body(h_ref, hj_ref, w1_ref, b1_ref, w2_ref, b2_ref, out_ref):
    hd = h_ref[...]
    w1 = w1_ref[...]
    b1 = b1_ref[...]
    w2 = w2_ref[...]
    b2 = b2_ref[...]
    for t in range(K):
        hs = hj_ref[t]
        e = jnp.concatenate([hs, hd], axis=1)
        pre = jnp.maximum(jnp.dot(e, w1, preferred_element_type=jnp.float32) + b1, 0.0)
        lg = jnp.dot(pre, w2, preferred_element_type=jnp.float32) + b2
        out_ref[:, t:t + 1] = jax.nn.sigmoid(lg)


def _final(h, hj, w1, b1, w2, b2, emb, hid):
    return pl.pallas_call(
        _final_body,
        grid=(GRID,),
        in_specs=[
            pl.BlockSpec((RB, emb), lambda i: (i, 0)),
            pl.BlockSpec((K, RB, emb), lambda i: (0, i, 0)),
            pl.BlockSpec((2 * emb, hid), lambda i: (0, 0)),
            pl.BlockSpec((1, hid), lambda i: (0, 0)),
            pl.BlockSpec((hid, 1), lambda i: (0, 0)),
            pl.BlockSpec((1, 1), lambda i: (0, 0)),
        ],
        out_specs=pl.BlockSpec((RB, K), lambda i: (i, 0)),
        out_shape=jax.ShapeDtypeStruct((N, K), jnp.float32),
    )(h, hj, w1, b1, w2, b2)


def _edge_conv_layer(xf, segs, brow, bcol, wa, ba, wb, bb):
    d = xf.shape[1]
    emb = wb.shape[1]
    idx = _knn(xf, xf.T, segs, brow, bcol, d)
    table = xf if d % 128 == 0 else jnp.concatenate(
        [xf, jnp.zeros((N, 128 - d % 128), jnp.float32)], axis=1)
    dg = table.shape[1]
    xj = _gather_rows(table, idx.T.reshape(-1))
    return _conv(xf, xj.reshape(K, N, dg), wa, ba.reshape(1, emb),
                 wb, bb.reshape(1, emb), d, dg, emb)


def kernel(x, batch, W1a, b1a, W1b, b1b, W2a, b2a, W2b, b2b, Wm1, bm1, Wm2, bm2):
    brow = batch.reshape(N, 1)
    bcol = batch.reshape(1, N)
    starts = jnp.searchsorted(batch, batch[::RB]).astype(jnp.int32)
    ends = jnp.searchsorted(batch, batch[RB - 1::RB], side='right').astype(jnp.int32)
    segs = jnp.concatenate([starts, ends])
    h = _edge_conv_layer(x, segs, brow, bcol, W1a, b1a, W1b, b1b)
    h = _edge_conv_layer(h, segs, brow, bcol, W2a, b2a, W2b, b2b)
    emb = h.shape[1]
    hid = Wm1.shape[1]
    idx = _knn(h, h.T, segs, brow, bcol, emb)
    hj = _gather_rows(h, idx.T.reshape(-1))
    probs = _final(h, hj.reshape(K, N, emb), Wm1, bm1.reshape(1, hid),
                   Wm2, bm2.reshape(1, 1), emb, hid)
    link_probs = probs.reshape(N * K, 1)
    src = idx.reshape(-1)
    dst = jnp.repeat(jnp.arange(N, dtype=jnp.int32), K)
    edge_index = jnp.stack([src, dst], axis=0)
    return link_probs, edge_index
